# baseline (speedup 1.0000x reference)
import functools

import jax
import jax.numpy as jnp
from jax import lax
from jax.experimental import pallas as pl
from jax.experimental.pallas import tpu as pltpu
from jax.experimental.pallas import tpu_sc as plsc

NC = 2
NS = 16
NW = NC * NS
K = 128

N_PAD = 10240
ROWS_PER_SUB = N_PAD // NS


def _sc_mesh():
    return plsc.VectorSubcoreMesh(
        core_axis_name="c", subcore_axis_name="s",
        num_cores=NC, num_subcores=NS)


def _make_deg_kernel(ch):
    @functools.partial(
        pl.kernel,
        out_type=jax.ShapeDtypeStruct((NC, N_PAD), jnp.float32),
        mesh=_sc_mesh(),
        scratch_types=[
            pltpu.VMEM((ch, K), jnp.int32),
            pltpu.VMEM((K,), jnp.float32),
            pltpu.VMEM_SHARED((N_PAD,), jnp.float32),
        ],
    )
    def deg_kernel(dst_hbm, zeros_hbm, out_hbm, dst_v, ones_v, deg_sh):
        c = lax.axis_index("c")
        s = lax.axis_index("s")
        wid = s * NC + c
        pltpu.sync_copy(dst_hbm.at[wid], dst_v)
        for i in range(K // 16):
            ones_v[pl.ds(i * 16, 16)] = jnp.ones((16,), jnp.float32)
        pltpu.sync_copy(zeros_hbm.at[pl.ds(0, ROWS_PER_SUB)],
                        deg_sh.at[pl.ds(s * ROWS_PER_SUB, ROWS_PER_SUB)])
        plsc.subcore_barrier()

        def body(j, carry):
            pltpu.sync_copy(ones_v, deg_sh.at[dst_v.at[j]], add=True)
            return carry

        lax.fori_loop(0, ch, body, 0)
        plsc.subcore_barrier()
        pltpu.sync_copy(deg_sh.at[pl.ds(s * ROWS_PER_SUB, ROWS_PER_SUB)],
                        out_hbm.at[c, pl.ds(s * ROWS_PER_SUB, ROWS_PER_SUB)])

    return deg_kernel


def _make_agg_kernel(ch, d):
    assert ch % 2 == 0

    @functools.partial(
        pl.kernel,
        out_type=jax.ShapeDtypeStruct((NC, N_PAD, d), jnp.float32),
        mesh=_sc_mesh(),
        scratch_types=[
            pltpu.VMEM((ch, K), jnp.int32),
            pltpu.VMEM((K,), jnp.int32),
            pltpu.VMEM((K,), jnp.int32),
            pltpu.VMEM((K, d), jnp.float32),
            pltpu.VMEM((K, d), jnp.float32),
            pltpu.VMEM_SHARED((N_PAD, d), jnp.float32),
            pltpu.SemaphoreType.DMA,
            pltpu.SemaphoreType.DMA,
            pltpu.SemaphoreType.DMA,
            pltpu.SemaphoreType.DMA,
        ],
    )
    def agg_kernel(y_hbm, src_hbm, dst_hbm, zeros_hbm, out_hbm,
                   src_v, d0, d1, b0, b1, z_sh, sg0, sg1, sd0, sd1):
        dbufs = (d0, d1)
        bufs = (b0, b1)
        gsems = (sg0, sg1)
        dsems = (sd0, sd1)
        c = lax.axis_index("c")
        s = lax.axis_index("s")
        wid = s * NC + c
        pltpu.sync_copy(src_hbm.at[wid], src_v)
        pltpu.sync_copy(zeros_hbm,
                        z_sh.at[pl.ds(s * ROWS_PER_SUB, ROWS_PER_SUB)])
        plsc.subcore_barrier()

        pltpu.async_copy(y_hbm.at[src_v.at[0]], bufs[0], gsems[0])
        pltpu.async_copy(dst_hbm.at[wid, 0], dbufs[0], dsems[0])

        def body(i, carry):
            for b in range(2):
                j = i * 2 + b
                jn = jnp.minimum(j + 1, ch - 1)
                pltpu.async_copy(y_hbm.at[src_v.at[jn]],
                                 bufs[1 - b], gsems[1 - b])
                pltpu.async_copy(dst_hbm.at[wid, jn],
                                 dbufs[1 - b], dsems[1 - b])
                pltpu.make_async_copy(y_hbm.at[src_v.at[0]],
                                      bufs[b], gsems[b]).wait()
                pltpu.make_async_copy(dst_hbm.at[wid, 0],
                                      dbufs[b], dsems[b]).wait()
                pltpu.sync_copy(bufs[b], z_sh.at[dbufs[b]], add=True)
            return carry

        lax.fori_loop(0, ch // 2, body, 0)
        pltpu.make_async_copy(y_hbm.at[src_v.at[0]],
                              bufs[0], gsems[0]).wait()
        pltpu.make_async_copy(dst_hbm.at[wid, 0],
                              dbufs[0], dsems[0]).wait()
        plsc.subcore_barrier()
        pltpu.sync_copy(z_sh.at[pl.ds(s * ROWS_PER_SUB, ROWS_PER_SUB)],
                        out_hbm.at[c, pl.ds(s * ROWS_PER_SUB, ROWS_PER_SUB)])

    return agg_kernel


def _dinv_from_parts(deg_parts_blk):
    deg = deg_parts_blk[:, 0] + deg_parts_blk[:, 1] + 1.0
    return lax.rsqrt(deg)


def _tc1_body(x_ref, w_ref, degp_ref, y_ref):
    dinv = _dinv_from_parts(degp_ref[...])
    h = jnp.dot(x_ref[...], w_ref[...], preferred_element_type=jnp.float32)
    y_ref[...] = h * dinv[:, None]


def _tc2_body(z_ref, y1_ref, degp_ref, b1_ref, w_ref, y2_ref):
    dinv = _dinv_from_parts(degp_ref[...])
    z = z_ref[0] + z_ref[1] + y1_ref[...]
    h = jax.nn.relu(z * dinv[:, None] + b1_ref[...])
    h2 = jnp.dot(h, w_ref[...], preferred_element_type=jnp.float32)
    y2_ref[...] = h2 * dinv[:, None]


def _tc3_body(z_ref, y2_ref, degp_ref, b_ref, out_ref):
    dinv = _dinv_from_parts(degp_ref[...])
    z = z_ref[0] + z_ref[1] + y2_ref[...]
    out_ref[...] = z * dinv[:, None] + b_ref[...]


def kernel(x, edge_index, W1, b1, Wmu, bmu, Wlv, blv):
    n, d_in = x.shape
    d_hid = W1.shape[1]
    d_lat = Wmu.shape[1]
    e = edge_index.shape[1]
    d_out2 = 2 * d_lat

    ch = -(-e // (NW * K))
    ch = -(-ch // 2) * 2
    e_pad = NW * ch * K
    pad = e_pad - e
    pad_dst = n + (jnp.arange(pad, dtype=jnp.int32) % (N_PAD - n))
    src_r = jnp.concatenate(
        [edge_index[0], jnp.zeros((pad,), jnp.int32)]).reshape(NW, ch, K)
    dst_r = jnp.concatenate(
        [edge_index[1], pad_dst]).reshape(NW, ch, K)

    zeros1 = jnp.zeros((ROWS_PER_SUB,), jnp.float32)
    zeros2 = jnp.zeros((ROWS_PER_SUB, d_hid), jnp.float32)

    Wcat = jnp.concatenate([Wmu, Wlv], axis=1)
    bcat = jnp.concatenate([bmu, blv]).reshape(1, d_out2)
    b1r = b1.reshape(1, d_hid)

    deg_parts = _make_deg_kernel(ch)(dst_r, zeros1).T

    BR = 1000
    g = n // BR
    row_spec = lambda d: pl.BlockSpec((BR, d), lambda i: (i, 0))
    part_spec = pl.BlockSpec((BR, NC), lambda i: (i, 0))
    zpart_spec = lambda d: pl.BlockSpec((NC, BR, d), lambda i: (0, i, 0))
    full_spec = lambda r, d: pl.BlockSpec((r, d), lambda i: (0, 0))

    y1 = pl.pallas_call(
        _tc1_body,
        grid=(g,),
        in_specs=[row_spec(d_in), full_spec(d_in, d_hid), part_spec],
        out_specs=row_spec(d_hid),
        out_shape=jax.ShapeDtypeStruct((n, d_hid), jnp.float32),
    )(x, W1, deg_parts)

    assert d_out2 == d_hid
    agg = _make_agg_kernel(ch, d_hid)
    z1_parts = agg(y1, src_r, dst_r, zeros2)

    y2 = pl.pallas_call(
        _tc2_body,
        grid=(g,),
        in_specs=[zpart_spec(d_hid), row_spec(d_hid), part_spec,
                  full_spec(1, d_hid), full_spec(d_hid, d_out2)],
        out_specs=row_spec(d_out2),
        out_shape=jax.ShapeDtypeStruct((n, d_out2), jnp.float32),
    )(z1_parts, y1, deg_parts, b1r, Wcat)

    z2_parts = agg(y2, src_r, dst_r, zeros2)

    out = pl.pallas_call(
        _tc3_body,
        grid=(g,),
        in_specs=[zpart_spec(d_out2), row_spec(d_out2), part_spec,
                  full_spec(1, d_out2)],
        out_specs=row_spec(d_out2),
        out_shape=jax.ShapeDtypeStruct((n, d_out2), jnp.float32),
    )(z2_parts, y2, deg_parts, bcat)

    return (out[:, :d_lat], out[:, d_lat:])


# speedup vs baseline: 1.8277x; 1.8277x over previous
import functools

import jax
import jax.numpy as jnp
from jax import lax
from jax.experimental import pallas as pl
from jax.experimental.pallas import tpu as pltpu
from jax.experimental.pallas import tpu_sc as plsc

NC = 2
NS = 16
NW = NC * NS
K = 128

N_PAD = 10240
ROWS_PER_SUB = N_PAD // NS


def _sc_mesh():
    return plsc.VectorSubcoreMesh(
        core_axis_name="c", subcore_axis_name="s",
        num_cores=NC, num_subcores=NS)


def _make_deg_kernel(ch, ch0, ch1):
    @functools.partial(
        pl.kernel,
        out_type=jax.ShapeDtypeStruct((NC, N_PAD), jnp.float32),
        mesh=_sc_mesh(),
        scratch_types=[
            pltpu.VMEM((ch, K), jnp.int32),
            pltpu.VMEM((K,), jnp.float32),
            pltpu.VMEM_SHARED((N_PAD,), jnp.float32),
        ],
    )
    def deg_kernel(dst_hbm, zeros_hbm, out_hbm, dst_v, ones_v, deg_sh):
        c = lax.axis_index("c")
        s = lax.axis_index("s")
        wid = s * NC + c
        pltpu.sync_copy(dst_hbm.at[wid], dst_v)
        for i in range(K // 16):
            ones_v[pl.ds(i * 16, 16)] = jnp.ones((16,), jnp.float32)
        pltpu.sync_copy(zeros_hbm.at[pl.ds(0, ROWS_PER_SUB)],
                        deg_sh.at[pl.ds(s * ROWS_PER_SUB, ROWS_PER_SUB)])
        plsc.subcore_barrier()

        def body(j, carry):
            pltpu.sync_copy(ones_v, deg_sh.at[dst_v.at[j]], add=True)
            return carry

        lax.fori_loop(0, jnp.where(c == 0, ch0, ch1), body, 0)
        plsc.subcore_barrier()
        pltpu.sync_copy(deg_sh.at[pl.ds(s * ROWS_PER_SUB, ROWS_PER_SUB)],
                        out_hbm.at[c, pl.ds(s * ROWS_PER_SUB, ROWS_PER_SUB)])

    return deg_kernel


def _make_agg_kernel(ch, ch0, ch1, d):
    @functools.partial(
        pl.kernel,
        out_type=jax.ShapeDtypeStruct((NC, N_PAD, d), jnp.float32),
        mesh=_sc_mesh(),
        scratch_types=[
            pltpu.VMEM((ch, K), jnp.int32),
            pltpu.VMEM((ch, K), jnp.int32),
            pltpu.VMEM((K, d), jnp.float32),
            pltpu.VMEM_SHARED((N_PAD, d), jnp.float32),
        ],
    )
    def agg_kernel(y_hbm, src_hbm, dst_hbm, zeros_hbm, out_hbm,
                   src_v, dst_v, buf, z_sh):
        c = lax.axis_index("c")
        s = lax.axis_index("s")
        wid = s * NC + c
        pltpu.sync_copy(src_hbm.at[wid], src_v)
        pltpu.sync_copy(dst_hbm.at[wid], dst_v)
        pltpu.sync_copy(zeros_hbm,
                        z_sh.at[pl.ds(s * ROWS_PER_SUB, ROWS_PER_SUB)])
        plsc.subcore_barrier()

        def body(j, carry):
            pltpu.sync_copy(y_hbm.at[src_v.at[j]], buf)
            pltpu.sync_copy(buf, z_sh.at[dst_v.at[j]], add=True)
            return carry

        lax.fori_loop(0, jnp.where(c == 0, ch0, ch1), body, 0)
        plsc.subcore_barrier()
        pltpu.sync_copy(z_sh.at[pl.ds(s * ROWS_PER_SUB, ROWS_PER_SUB)],
                        out_hbm.at[c, pl.ds(s * ROWS_PER_SUB, ROWS_PER_SUB)])

    return agg_kernel


def _dinv_from_parts(deg_parts_blk):
    deg = deg_parts_blk[:, 0] + deg_parts_blk[:, 1] + 1.0
    return lax.rsqrt(deg)


def _tc1_body(x_ref, w_ref, degp_ref, y_ref):
    dinv = _dinv_from_parts(degp_ref[...])
    h = jnp.dot(x_ref[...], w_ref[...], preferred_element_type=jnp.float32)
    y_ref[...] = h * dinv[:, None]


def _tc2_body(z_ref, y1_ref, degp_ref, b1_ref, w_ref, y2_ref):
    dinv = _dinv_from_parts(degp_ref[...])
    z = z_ref[0] + z_ref[1] + y1_ref[...]
    h = jax.nn.relu(z * dinv[:, None] + b1_ref[...])
    h2 = jnp.dot(h, w_ref[...], preferred_element_type=jnp.float32)
    y2_ref[...] = h2 * dinv[:, None]


def _tc3_body(z_ref, y2_ref, degp_ref, b_ref, out_ref):
    dinv = _dinv_from_parts(degp_ref[...])
    z = z_ref[0] + z_ref[1] + y2_ref[...]
    out_ref[...] = z * dinv[:, None] + b_ref[...]


def kernel(x, edge_index, W1, b1, Wmu, bmu, Wlv, blv):
    n, d_in = x.shape
    d_hid = W1.shape[1]
    d_lat = Wmu.shape[1]
    e = edge_index.shape[1]
    d_out2 = 2 * d_lat

    e0 = (e * 5) // 8
    m0 = -(-e0 // NS)
    m1 = -(-(e - e0) // NS)
    ch0 = -(-m0 // K)
    ch1 = -(-m1 // K)
    ch = max(ch0, ch1)

    def _dummy_rows(m):
        return n + (jnp.arange(m, dtype=jnp.int32) % (N_PAD - n))

    def _part(row, fill_fn):
        def side(seg):
            m = -(-seg.shape[0] // NS)
            s1 = jnp.concatenate(
                [seg, fill_fn(NS * m - seg.shape[0])]).reshape(NS, m)
            s2 = fill_fn(NS * (ch * K - m)).reshape(NS, ch * K - m)
            return jnp.concatenate([s1, s2], axis=1).reshape(NS, ch, K)

        a = side(row[:e0])
        b = side(row[e0:])
        return jnp.stack([a, b], axis=1).reshape(NW, ch, K)

    src_r = _part(edge_index[0], lambda m: jnp.zeros((m,), jnp.int32))
    dst_r = _part(edge_index[1], _dummy_rows)

    zeros1 = jnp.zeros((ROWS_PER_SUB,), jnp.float32)
    zeros2 = jnp.zeros((ROWS_PER_SUB, d_hid), jnp.float32)

    Wcat = jnp.concatenate([Wmu, Wlv], axis=1)
    bcat = jnp.concatenate([bmu, blv]).reshape(1, d_out2)
    b1r = b1.reshape(1, d_hid)

    deg_parts = _make_deg_kernel(ch, ch0, ch1)(dst_r, zeros1).T

    BR = 1000
    g = n // BR
    row_spec = lambda d: pl.BlockSpec((BR, d), lambda i: (i, 0))
    part_spec = pl.BlockSpec((BR, NC), lambda i: (i, 0))
    zpart_spec = lambda d: pl.BlockSpec((NC, BR, d), lambda i: (0, i, 0))
    full_spec = lambda r, d: pl.BlockSpec((r, d), lambda i: (0, 0))

    y1 = pl.pallas_call(
        _tc1_body,
        grid=(g,),
        in_specs=[row_spec(d_in), full_spec(d_in, d_hid), part_spec],
        out_specs=row_spec(d_hid),
        out_shape=jax.ShapeDtypeStruct((n, d_hid), jnp.float32),
    )(x, W1, deg_parts)

    assert d_out2 == d_hid
    agg = _make_agg_kernel(ch, ch0, ch1, d_hid)
    z1_parts = agg(y1, src_r, dst_r, zeros2)

    y2 = pl.pallas_call(
        _tc2_body,
        grid=(g,),
        in_specs=[zpart_spec(d_hid), row_spec(d_hid), part_spec,
                  full_spec(1, d_hid), full_spec(d_hid, d_out2)],
        out_specs=row_spec(d_out2),
        out_shape=jax.ShapeDtypeStruct((n, d_out2), jnp.float32),
    )(z1_parts, y1, deg_parts, b1r, Wcat)

    z2_parts = agg(y2, src_r, dst_r, zeros2)

    out = pl.pallas_call(
        _tc3_body,
        grid=(g,),
        in_specs=[zpart_spec(d_out2), row_spec(d_out2), part_spec,
                  full_spec(1, d_out2)],
        out_specs=row_spec(d_out2),
        out_shape=jax.ShapeDtypeStruct((n, d_out2), jnp.float32),
    )(z2_parts, y2, deg_parts, bcat)

    return (out[:, :d_lat], out[:, d_lat:])
